# baseline (speedup 1.0000x reference)
import jax
import jax.numpy as jnp
from jax.experimental import pallas as pl

_BM = 4096


def _split_cat(x):
    hi = x.astype(jnp.bfloat16)
    lo = (x - hi.astype(jnp.float32)).astype(jnp.bfloat16)
    return jnp.concatenate([hi, lo], axis=1)


def _layer(x, w_ref, b_ref):
    n = b_ref.shape[1]
    r = jnp.dot(_split_cat(x), w_ref[...], preferred_element_type=jnp.float32)
    return jnp.tanh(r[:, :n] + r[:, n:] + b_ref[...])


def _ffn_block(obs0_ref, obs1_ref, w1_ref, w2_ref, b1_ref, b2_ref, out_ref):
    out_ref[0] = _layer(_layer(obs0_ref[...], w1_ref, b1_ref), w2_ref, b2_ref)
    out_ref[1] = _layer(_layer(obs1_ref[...], w1_ref, b1_ref), w2_ref, b2_ref)


def _pack_weights(w):
    hi = w.astype(jnp.bfloat16)
    lo = (w - hi.astype(jnp.float32)).astype(jnp.bfloat16)
    half = jnp.concatenate([hi, lo], axis=1)
    return jnp.concatenate([half, half], axis=0)


def kernel(obs, W1, W2, b1, b2):
    if obs.ndim == 1:
        obs = obs[None, :]
    batch, n_in = obs.shape
    n_hid = W1.shape[1]
    n_out = W2.shape[1]
    w1p = _pack_weights(W1)
    w2p = _pack_weights(W2)
    bm = min(_BM, batch // 2)
    steps = pl.cdiv(batch // 2, bm)
    rep = lambda i: (0, 0)
    out = pl.pallas_call(
        _ffn_block,
        grid=(steps,),
        in_specs=[
            pl.BlockSpec((bm, n_in), lambda i: (i, 0)),
            pl.BlockSpec((bm, n_in), lambda i, s=steps: (i + s, 0)),
            pl.BlockSpec((2 * n_in, 2 * n_hid), rep),
            pl.BlockSpec((2 * n_hid, 2 * n_out), rep),
            pl.BlockSpec((1, n_hid), rep),
            pl.BlockSpec((1, n_out), rep),
        ],
        out_specs=pl.BlockSpec((2, bm, n_out), lambda i: (0, i, 0)),
        out_shape=jax.ShapeDtypeStruct((2, batch // 2, n_out), jnp.float32),
    )(obs, obs, w1p, w2p, b1[None, :], b2[None, :])
    return out.reshape(batch, n_out)
